# baseline (speedup 1.0000x reference)
import functools
import jax
import jax.numpy as jnp
from jax.experimental import pallas as pl

_EPS = 1e-12
_W_K = 12.0
_LAYERS = 2
_BN = 1024


def _score_body(sess_ref, emb_ref, out_ref):
    sess = sess_ref[...]
    sn = jnp.sqrt(jnp.sum(sess * sess, axis=-1, keepdims=True))
    sel = _W_K * sess / jnp.maximum(sn, _EPS)
    emb = emb_ref[...]
    en = jnp.sqrt(jnp.sum(emb * emb, axis=-1, keepdims=True))
    embn = emb / jnp.maximum(en, _EPS)
    out_ref[...] = jax.lax.dot_general(
        sel, embn, (((1,), (1,)), ((), ())),
        preferred_element_type=jnp.float32)


def _scores(sess_emb, graph_item_embs):
    b, d = sess_emb.shape
    n = graph_item_embs.shape[0]
    grid = (n + _BN - 1) // _BN
    return pl.pallas_call(
        _score_body,
        grid=(grid,),
        in_specs=[
            pl.BlockSpec((b, d), lambda i: (0, 0)),
            pl.BlockSpec((_BN, d), lambda i: (i, 0)),
        ],
        out_specs=pl.BlockSpec((b, _BN), lambda i: (0, i)),
        out_shape=jax.ShapeDtypeStruct((b, n), jnp.float32),
    )(sess_emb, graph_item_embs)


def kernel(item_embedding, edge_val, items, inputs, alias_inputs, edge_row, edge_col):
    n = item_embedding.shape[0]

    degrees = jax.ops.segment_sum(edge_val, edge_row, num_segments=n)
    safe_deg = jnp.where(degrees > 0, degrees, 1.0)
    vals = edge_val / safe_deg[edge_row]

    h = item_embedding
    acc = item_embedding
    for _ in range(_LAYERS):
        msg = vals[:, None] * h[edge_col]
        h = jax.ops.segment_sum(msg, edge_row, num_segments=n)
        acc = acc + h
    graph_item_embs = acc / (_LAYERS + 1)

    hidden = graph_item_embs[items]
    idx = jnp.broadcast_to(alias_inputs[:, :, None], hidden.shape)
    seq_hidden = jnp.take_along_axis(hidden, idx, axis=1)

    mask = (inputs != 0)[:, :, None].astype(seq_hidden.dtype)
    sess_emb = jnp.sum(seq_hidden * mask, axis=1) / jnp.sum(mask, axis=1)

    return _scores(sess_emb, graph_item_embs)


# speedup vs baseline: 5.2599x; 5.2599x over previous
import functools
import jax
import jax.numpy as jnp
from jax import lax
from jax.experimental import pallas as pl
from jax.experimental.pallas import tpu as pltpu
from jax.experimental.pallas import tpu_sc as plsc

_EPS = 1e-12
_W_K = 12.0
_D = 32
_N_PAD = 100352
_H = _N_PAD // 2
_ACC_R = _H + 8
_TRASH = _H
_PW = _H // 16
_CH = 128
_BN = 1024


def _mesh():
    return plsc.VectorSubcoreMesh(core_axis_name="c", subcore_axis_name="s")


def _iota16():
    return lax.iota(jnp.int32, 16)


def _adjust_rows(rowi, half_base):
    for q in range(8):
        sl = pl.ds(q * 16, 16)
        r = rowi[0, sl]
        local = r - half_base
        oob = (local < 0) | (local >= _H)
        rowi[0, sl] = jnp.where(oob, _TRASH, local)


def _degrees(edge_row, edge_val):
    e = edge_row.shape[0]
    n_chunks = e // _CH

    @functools.partial(
        pl.kernel,
        mesh=_mesh(),
        out_type=jax.ShapeDtypeStruct((_N_PAD,), jnp.float32),
        compiler_params=pltpu.CompilerParams(use_tc_tiling_on_sc=False, needs_layout_passes=False),
        scratch_types=[
            pltpu.VMEM((1, _CH), jnp.int32),
            pltpu.VMEM((_CH,), jnp.float32),
            pltpu.VMEM((_PW,), jnp.float32),
            pltpu.VMEM_SHARED((_ACC_R,), jnp.float32),
        ],
    )
    def deg_k(erow, eval_, dout, rowi, valv, zbuf, dacc):
        cid = lax.axis_index("c")
        sid = lax.axis_index("s")
        half_base = cid * _H
        base = sid * _PW
        zf = jnp.zeros((16,), jnp.float32)

        def zero_body(q, _):
            zbuf[pl.ds(q * 16, 16)] = zf
            return 0
        lax.fori_loop(0, _PW // 16, zero_body, 0)
        pltpu.sync_copy(zbuf, dacc.at[pl.ds(base, _PW)])

        @pl.when(sid == 0)
        def _():
            pltpu.sync_copy(zbuf.at[pl.ds(0, 8)], dacc.at[pl.ds(_TRASH, 8)])

        plsc.subcore_barrier()

        n_w = (n_chunks - sid + 15) // 16

        def chunk_body(c, _):
            ebase = (sid + c * 16) * _CH
            pltpu.sync_copy(erow.at[pl.ds(ebase, _CH)], rowi.at[0])
            pltpu.sync_copy(eval_.at[pl.ds(ebase, _CH)], valv)
            _adjust_rows(rowi, half_base)
            pltpu.sync_copy(valv, dacc.at[rowi.at[0]], add=True)
            return 0
        lax.fori_loop(0, n_w, chunk_body, 0)

        plsc.subcore_barrier()
        pltpu.sync_copy(dacc.at[pl.ds(base, _PW)], zbuf)
        pltpu.sync_copy(zbuf, dout.at[pl.ds(half_base + base, _PW)])

    return deg_k(edge_row, edge_val)


def _conv_layer(h, degrees, edge_row, edge_col, edge_val):
    e = edge_row.shape[0]
    n_chunks = e // _CH

    @functools.partial(
        pl.kernel,
        mesh=_mesh(),
        out_type=jax.ShapeDtypeStruct((_N_PAD, _D), jnp.float32),
        compiler_params=pltpu.CompilerParams(use_tc_tiling_on_sc=False, needs_layout_passes=False),
        scratch_types=[
            pltpu.VMEM((1, _CH), jnp.int32),
            pltpu.VMEM((_CH,), jnp.int32),
            pltpu.VMEM((_CH,), jnp.float32),
            pltpu.VMEM((_CH, _D), jnp.float32),
            pltpu.VMEM((512, _D), jnp.float32),
            pltpu.VMEM((512,), jnp.float32),
            pltpu.VMEM_SHARED((_ACC_R, _D), jnp.float32),
            pltpu.SemaphoreType.DMA,
        ],
    )
    def conv_k(h_hbm, deg_hbm, erow, ecol, eval_, hout,
               rowi, coli, valv, rowsb, workb, degb, acc, sem):
        cid = lax.axis_index("c")
        sid = lax.axis_index("s")
        half_base = cid * _H
        base = sid * _PW
        iota = _iota16()
        zf = jnp.zeros((16,), jnp.float32)

        def zrow(r, _):
            workb[r, pl.ds(0, 16)] = zf
            workb[r, pl.ds(16, 16)] = zf
            return 0
        lax.fori_loop(0, 512, zrow, 0)
        for off, sz in ((0, 512), (512, 512), (1024, 512), (1536, 512), (2048, 512), (2560, 512), (3072, 64)):
            pltpu.sync_copy(workb.at[pl.ds(0, sz)],
                            acc.at[pl.ds(base + off, sz)])

        @pl.when(sid == 0)
        def _():
            pltpu.sync_copy(workb.at[pl.ds(0, 8)], acc.at[pl.ds(_TRASH, 8)])

        plsc.subcore_barrier()

        n_w = (n_chunks - sid + 15) // 16

        def chunk_body(c, _):
            ebase = (sid + c * 16) * _CH
            pltpu.sync_copy(erow.at[pl.ds(ebase, _CH)], rowi.at[0])
            pltpu.sync_copy(ecol.at[pl.ds(ebase, _CH)], coli)
            pltpu.sync_copy(eval_.at[pl.ds(ebase, _CH)], valv)
            pltpu.async_copy(h_hbm.at[coli], rowsb, sem).wait()
            for q in range(8):
                sl = pl.ds(q * 16, 16)
                r = rowi[0, sl]
                local = r - half_base
                oob = (local < 0) | (local >= _H)
                rowi[0, sl] = jnp.where(oob, _TRASH, local)
                v = valv[sl]
                for j in range(16):
                    e_i = q * 16 + j
                    wv = jnp.broadcast_to(v[j], (16,))
                    rowsb[e_i, pl.ds(0, 16)] = rowsb[e_i, pl.ds(0, 16)] * wv
                    rowsb[e_i, pl.ds(16, 16)] = rowsb[e_i, pl.ds(16, 16)] * wv
            pltpu.sync_copy(rowsb, acc.at[rowi.at[0]], add=True)
            return 0
        lax.fori_loop(0, n_w, chunk_body, 0)

        plsc.subcore_barrier()

        go = half_base + base
        for off, sz in ((0, 512), (512, 512), (1024, 512), (1536, 512), (2048, 512), (2560, 512), (3072, 64)):
            pltpu.sync_copy(acc.at[pl.ds(base + off, sz)],
                            workb.at[pl.ds(0, sz)])
            pltpu.sync_copy(deg_hbm.at[pl.ds(go + off, sz)],
                            degb.at[pl.ds(0, sz)])

            def scale_body(q, _):
                dv = degb[pl.ds(q * 16, 16)]
                w = 1.0 / jnp.where(dv > 0, dv, 1.0)
                for j in range(16):
                    r_i = q * 16 + j
                    wv = jnp.broadcast_to(w[j], (16,))
                    workb[r_i, pl.ds(0, 16)] = workb[r_i, pl.ds(0, 16)] * wv
                    workb[r_i, pl.ds(16, 16)] = workb[r_i, pl.ds(16, 16)] * wv
                return 0
            lax.fori_loop(0, sz // 16, scale_body, 0)
            pltpu.sync_copy(workb.at[pl.ds(0, sz)],
                            hout.at[pl.ds(go + off, sz)])

    return conv_k(h, degrees, edge_row, edge_col, edge_val)


def _session_emb(h0, h1, h2, items, inputs, alias_inputs):
    b, l = items.shape
    per_w = b // 32

    @functools.partial(
        pl.kernel,
        mesh=_mesh(),
        out_type=jax.ShapeDtypeStruct((b, _D), jnp.float32),
        compiler_params=pltpu.CompilerParams(use_tc_tiling_on_sc=False, needs_layout_passes=False),
        scratch_types=[
            pltpu.VMEM((per_w * l,), jnp.int32),
            pltpu.VMEM((per_w * l,), jnp.int32),
            pltpu.VMEM((per_w * l,), jnp.int32),
            pltpu.VMEM((64,), jnp.int32),
            pltpu.VMEM((64,), jnp.float32),
            pltpu.VMEM((192, _D), jnp.float32),
            pltpu.VMEM((per_w, _D), jnp.float32),
            pltpu.SemaphoreType.DMA,
        ],
    )
    def sess_k(t0, t1, t2, items_h, inputs_h, alias_h, sout,
               itemsv, inputsv, aliasv, idxb, wb, rowsb, sessb, sem):
        cid = lax.axis_index("c")
        sid = lax.axis_index("s")
        wid = sid * 2 + cid
        b0 = wid * per_w
        pltpu.sync_copy(items_h.at[pl.ds(b0 * l, per_w * l)], itemsv)
        pltpu.sync_copy(inputs_h.at[pl.ds(b0 * l, per_w * l)], inputsv)
        pltpu.sync_copy(alias_h.at[pl.ds(b0 * l, per_w * l)], aliasv)
        iota = _iota16()
        tailw = jnp.where(iota >= 14, 1.0, 0.0)

        def body(s, _):
            fb = s * l
            fbvec = jnp.broadcast_to(fb, (16,))
            for g in range(4):
                soff = g * 16 if g < 3 else 34
                dst = pl.ds(g * 16, 16)
                a = aliasv[pl.ds(fb + soff, 16)]
                idxb[dst] = plsc.load_gather(itemsv, [fbvec + a])
                iv = inputsv[pl.ds(fb + soff, 16)]
                w = jnp.where(iv != 0, 1.0, 0.0)
                if g == 3:
                    w = w * tailw
                wb[dst] = w
            wt = (wb[pl.ds(0, 16)] + wb[pl.ds(16, 16)]
                  + wb[pl.ds(32, 16)] + wb[pl.ds(48, 16)])
            cnt = jnp.sum(wt)
            copies = [
                pltpu.async_copy(tbl.at[idxb], rowsb.at[pl.ds(64 * k, 64)], sem)
                for k, tbl in enumerate((t0, t1, t2))
            ]
            for c in copies:
                c.wait()

            def accum_g(gi, carry):
                a0c, a1c = carry
                wg = wb[pl.ds((gi % 4) * 16, 16)]
                for j in range(16):
                    r_i = gi * 16 + j
                    wv = jnp.broadcast_to(wg[j], (16,))
                    a0c = a0c + rowsb[r_i, pl.ds(0, 16)] * wv
                    a1c = a1c + rowsb[r_i, pl.ds(16, 16)] * wv
                return (a0c, a1c)
            acc0, acc1 = lax.fori_loop(
                0, 12, accum_g,
                (jnp.zeros((16,), jnp.float32), jnp.zeros((16,), jnp.float32)))
            scalev = 1.0 / (3.0 * jnp.broadcast_to(cnt, (16,)))
            sessb[s, pl.ds(0, 16)] = acc0 * scalev
            sessb[s, pl.ds(16, 16)] = acc1 * scalev
            return 0
        lax.fori_loop(0, per_w, body, 0)
        pltpu.sync_copy(sessb, sout.at[pl.ds(b0, per_w)])

    return sess_k(h0, h1, h2, items.reshape(-1), inputs.reshape(-1),
                  alias_inputs.reshape(-1))


def _score_body(sess_ref, e0_ref, e1_ref, e2_ref, out_ref):
    sess = sess_ref[...]
    sn = jnp.sqrt(jnp.sum(sess * sess, axis=-1, keepdims=True))
    sel = _W_K * sess / jnp.maximum(sn, _EPS)
    emb = (e0_ref[...] + e1_ref[...] + e2_ref[...]) * (1.0 / 3.0)
    en = jnp.sqrt(jnp.sum(emb * emb, axis=-1, keepdims=True))
    embn = emb / jnp.maximum(en, _EPS)
    out_ref[...] = lax.dot_general(
        sel, embn, (((1,), (1,)), ((), ())),
        preferred_element_type=jnp.float32)


def _scores(sess_emb, h0, h1, h2, n):
    b, d = sess_emb.shape
    grid = _N_PAD // _BN
    return pl.pallas_call(
        _score_body,
        grid=(grid,),
        in_specs=[
            pl.BlockSpec((b, d), lambda i: (0, 0)),
            pl.BlockSpec((_BN, d), lambda i: (i, 0)),
            pl.BlockSpec((_BN, d), lambda i: (i, 0)),
            pl.BlockSpec((_BN, d), lambda i: (i, 0)),
        ],
        out_specs=pl.BlockSpec((b, _BN), lambda i: (0, i)),
        out_shape=jax.ShapeDtypeStruct((b, n), jnp.float32),
    )(sess_emb, h0, h1, h2)


def kernel(item_embedding, edge_val, items, inputs, alias_inputs, edge_row, edge_col):
    n = item_embedding.shape[0]
    h0 = jnp.pad(item_embedding, ((0, _N_PAD - n), (0, 0)))
    degrees = _degrees(edge_row, edge_val)
    h1 = _conv_layer(h0, degrees, edge_row, edge_col, edge_val)
    h2 = _conv_layer(h1, degrees, edge_row, edge_col, edge_val)
    sess_emb = _session_emb(h0, h1, h2, items, inputs, alias_inputs)
    return _scores(sess_emb, h0, h1, h2, n)


# speedup vs baseline: 8.1310x; 1.5459x over previous
import functools
import jax
import jax.numpy as jnp
from jax import lax
from jax.experimental import pallas as pl
from jax.experimental.pallas import tpu as pltpu
from jax.experimental.pallas import tpu_sc as plsc

_EPS = 1e-12
_W_K = 12.0
_D = 32
_N_PAD = 100352
_H = _N_PAD // 2
_ACC_R = _H + 8
_TRASH = _H
_PW = _H // 16
_CH = 128
_BN = 1024


def _mesh():
    return plsc.VectorSubcoreMesh(core_axis_name="c", subcore_axis_name="s")


def _iota16():
    return lax.iota(jnp.int32, 16)


def _adjust_rows(rowi, half_base):
    for q in range(8):
        sl = pl.ds(q * 16, 16)
        r = rowi[0, sl]
        local = r - half_base
        oob = (local < 0) | (local >= _H)
        rowi[0, sl] = jnp.where(oob, _TRASH, local)


def _degrees(edge_row, edge_val):
    e = edge_row.shape[0]
    n_chunks = e // _CH

    @functools.partial(
        pl.kernel,
        mesh=_mesh(),
        out_type=jax.ShapeDtypeStruct((_N_PAD,), jnp.float32),
        compiler_params=pltpu.CompilerParams(use_tc_tiling_on_sc=False, needs_layout_passes=False),
        scratch_types=[
            pltpu.VMEM((1, _CH), jnp.int32),
            pltpu.VMEM((_CH,), jnp.float32),
            pltpu.VMEM((_PW,), jnp.float32),
            pltpu.VMEM_SHARED((_ACC_R,), jnp.float32),
        ],
    )
    def deg_k(erow, eval_, dout, rowi, valv, zbuf, dacc):
        cid = lax.axis_index("c")
        sid = lax.axis_index("s")
        half_base = cid * _H
        base = sid * _PW
        zf = jnp.zeros((16,), jnp.float32)

        def zero_body(q, _):
            zbuf[pl.ds(q * 16, 16)] = zf
            return 0
        lax.fori_loop(0, _PW // 16, zero_body, 0)
        pltpu.sync_copy(zbuf, dacc.at[pl.ds(base, _PW)])

        @pl.when(sid == 0)
        def _():
            pltpu.sync_copy(zbuf.at[pl.ds(0, 8)], dacc.at[pl.ds(_TRASH, 8)])

        plsc.subcore_barrier()

        n_w = (n_chunks - sid + 15) // 16

        def chunk_body(c, _):
            ebase = (sid + c * 16) * _CH
            pltpu.sync_copy(erow.at[pl.ds(ebase, _CH)], rowi.at[0])
            pltpu.sync_copy(eval_.at[pl.ds(ebase, _CH)], valv)
            _adjust_rows(rowi, half_base)
            pltpu.sync_copy(valv, dacc.at[rowi.at[0]], add=True)
            return 0
        lax.fori_loop(0, n_w, chunk_body, 0)

        plsc.subcore_barrier()
        pltpu.sync_copy(dacc.at[pl.ds(base, _PW)], zbuf)
        pltpu.sync_copy(zbuf, dout.at[pl.ds(half_base + base, _PW)])

    return deg_k(edge_row, edge_val)


def _conv_layer(h, degrees, edge_row, edge_col, edge_val):
    e = edge_row.shape[0]
    n_chunks = e // _CH

    @functools.partial(
        pl.kernel,
        mesh=_mesh(),
        out_type=jax.ShapeDtypeStruct((_N_PAD, _D), jnp.float32),
        compiler_params=pltpu.CompilerParams(use_tc_tiling_on_sc=False, needs_layout_passes=False),
        scratch_types=[
            pltpu.VMEM((1, _CH), jnp.int32),
            pltpu.VMEM((_CH,), jnp.int32),
            pltpu.VMEM((_CH,), jnp.float32),
            pltpu.VMEM((_CH, _D), jnp.float32),
            pltpu.VMEM((1, _CH), jnp.int32),
            pltpu.VMEM((_CH,), jnp.int32),
            pltpu.VMEM((_CH,), jnp.float32),
            pltpu.VMEM((_CH, _D), jnp.float32),
            pltpu.VMEM((256, _D), jnp.float32),
            pltpu.VMEM((256,), jnp.float32),
            pltpu.VMEM_SHARED((_ACC_R, _D), jnp.float32),
            pltpu.SemaphoreType.DMA,
            pltpu.SemaphoreType.DMA,
            pltpu.SemaphoreType.DMA,
        ],
    )
    def conv_k(h_hbm, deg_hbm, erow, ecol, eval_, hout,
               rowi, coli, valv, rowsb, rowi2, coli2, valv2, rowsb2,
               workb, degb, acc, sem_l, sem_a, sem_b):
        cid = lax.axis_index("c")
        sid = lax.axis_index("s")
        half_base = cid * _H
        base = sid * _PW
        iota = _iota16()
        zf = jnp.zeros((16,), jnp.float32)

        def zrow(r, _):
            workb[r, pl.ds(0, 16)] = zf
            workb[r, pl.ds(16, 16)] = zf
            return 0
        lax.fori_loop(0, 256, zrow, 0)
        for k in range(12):
            pltpu.sync_copy(workb, acc.at[pl.ds(base + k * 256, 256)])
        pltpu.sync_copy(workb.at[pl.ds(0, 64)], acc.at[pl.ds(base + 3072, 64)])

        @pl.when(sid == 0)
        def _():
            pltpu.sync_copy(workb.at[pl.ds(0, 8)], acc.at[pl.ds(_TRASH, 8)])

        plsc.subcore_barrier()

        n_w = (n_chunks - sid + 15) // 16

        def scale_chunk(rowi_r, valv_r, rowsb_r):
            for q in range(8):
                sl = pl.ds(q * 16, 16)
                r = rowi_r[0, sl]
                local = r - half_base
                oob = (local < 0) | (local >= _H)
                rowi_r[0, sl] = jnp.where(oob, _TRASH, local)
                v = valv_r[sl]
                for j in range(16):
                    e_i = q * 16 + j
                    wv = jnp.broadcast_to(v[j], (16,))
                    rowsb_r[e_i, pl.ds(0, 16)] = rowsb_r[e_i, pl.ds(0, 16)] * wv
                    rowsb_r[e_i, pl.ds(16, 16)] = (
                        rowsb_r[e_i, pl.ds(16, 16)] * wv)

        def load_edges(c, rowi_r, coli_r, valv_r):
            ebase = (sid + c * 16) * _CH
            return [
                pltpu.async_copy(erow.at[pl.ds(ebase, _CH)], rowi_r.at[0],
                                 sem_l),
                pltpu.async_copy(ecol.at[pl.ds(ebase, _CH)], coli_r, sem_l),
                pltpu.async_copy(eval_.at[pl.ds(ebase, _CH)], valv_r, sem_l),
            ]

        def pair_body(p, _):
            copies = load_edges(2 * p, rowi, coli, valv)
            copies += load_edges(2 * p + 1, rowi2, coli2, valv2)
            for cp in copies:
                cp.wait()
            ga = pltpu.async_copy(h_hbm.at[coli], rowsb, sem_a)
            gb = pltpu.async_copy(h_hbm.at[coli2], rowsb2, sem_b)
            ga.wait()
            scale_chunk(rowi, valv, rowsb)
            pltpu.sync_copy(rowsb, acc.at[rowi.at[0]], add=True)
            gb.wait()
            scale_chunk(rowi2, valv2, rowsb2)
            pltpu.sync_copy(rowsb2, acc.at[rowi2.at[0]], add=True)
            return 0
        lax.fori_loop(0, n_w // 2, pair_body, 0)

        @pl.when(n_w % 2 == 1)
        def _():
            c = n_w - 1
            for cp in load_edges(c, rowi, coli, valv):
                cp.wait()
            pltpu.async_copy(h_hbm.at[coli], rowsb, sem_a).wait()
            scale_chunk(rowi, valv, rowsb)
            pltpu.sync_copy(rowsb, acc.at[rowi.at[0]], add=True)

        plsc.subcore_barrier()

        go = half_base + base
        for off, sz in tuple((k * 256, 256) for k in range(12)) + ((3072, 64),):
            pltpu.sync_copy(acc.at[pl.ds(base + off, sz)],
                            workb.at[pl.ds(0, sz)])
            pltpu.sync_copy(deg_hbm.at[pl.ds(go + off, sz)],
                            degb.at[pl.ds(0, sz)])

            def scale_body(q, _):
                dv = degb[pl.ds(q * 16, 16)]
                w = 1.0 / jnp.where(dv > 0, dv, 1.0)
                for j in range(16):
                    r_i = q * 16 + j
                    wv = jnp.broadcast_to(w[j], (16,))
                    workb[r_i, pl.ds(0, 16)] = workb[r_i, pl.ds(0, 16)] * wv
                    workb[r_i, pl.ds(16, 16)] = workb[r_i, pl.ds(16, 16)] * wv
                return 0
            lax.fori_loop(0, sz // 16, scale_body, 0)
            pltpu.sync_copy(workb.at[pl.ds(0, sz)],
                            hout.at[pl.ds(go + off, sz)])

    return conv_k(h, degrees, edge_row, edge_col, edge_val)


def _session_emb(h0, h1, h2, items, inputs, alias_inputs):
    b, l = items.shape
    per_w = b // 32

    @functools.partial(
        pl.kernel,
        mesh=_mesh(),
        out_type=jax.ShapeDtypeStruct((b, _D), jnp.float32),
        compiler_params=pltpu.CompilerParams(use_tc_tiling_on_sc=False, needs_layout_passes=False),
        scratch_types=[
            pltpu.VMEM((per_w * l,), jnp.int32),
            pltpu.VMEM((per_w * l,), jnp.int32),
            pltpu.VMEM((per_w * l,), jnp.int32),
            pltpu.VMEM((64,), jnp.int32),
            pltpu.VMEM((64,), jnp.float32),
            pltpu.VMEM((192, _D), jnp.float32),
            pltpu.VMEM((per_w, _D), jnp.float32),
            pltpu.SemaphoreType.DMA,
        ],
    )
    def sess_k(t0, t1, t2, items_h, inputs_h, alias_h, sout,
               itemsv, inputsv, aliasv, idxb, wb, rowsb, sessb, sem):
        cid = lax.axis_index("c")
        sid = lax.axis_index("s")
        wid = sid * 2 + cid
        b0 = wid * per_w
        pltpu.sync_copy(items_h.at[pl.ds(b0 * l, per_w * l)], itemsv)
        pltpu.sync_copy(inputs_h.at[pl.ds(b0 * l, per_w * l)], inputsv)
        pltpu.sync_copy(alias_h.at[pl.ds(b0 * l, per_w * l)], aliasv)
        iota = _iota16()
        tailw = jnp.where(iota >= 14, 1.0, 0.0)

        def body(s, _):
            fb = s * l
            fbvec = jnp.broadcast_to(fb, (16,))
            for g in range(4):
                soff = g * 16 if g < 3 else 34
                dst = pl.ds(g * 16, 16)
                a = aliasv[pl.ds(fb + soff, 16)]
                idxb[dst] = plsc.load_gather(itemsv, [fbvec + a])
                iv = inputsv[pl.ds(fb + soff, 16)]
                w = jnp.where(iv != 0, 1.0, 0.0)
                if g == 3:
                    w = w * tailw
                wb[dst] = w
            wt = (wb[pl.ds(0, 16)] + wb[pl.ds(16, 16)]
                  + wb[pl.ds(32, 16)] + wb[pl.ds(48, 16)])
            cnt = jnp.sum(wt)
            copies = [
                pltpu.async_copy(tbl.at[idxb], rowsb.at[pl.ds(64 * k, 64)], sem)
                for k, tbl in enumerate((t0, t1, t2))
            ]
            for c in copies:
                c.wait()

            def accum_g(gi, carry):
                a0c, a1c = carry
                wg = wb[pl.ds((gi % 4) * 16, 16)]
                for j in range(16):
                    r_i = gi * 16 + j
                    wv = jnp.broadcast_to(wg[j], (16,))
                    a0c = a0c + rowsb[r_i, pl.ds(0, 16)] * wv
                    a1c = a1c + rowsb[r_i, pl.ds(16, 16)] * wv
                return (a0c, a1c)
            acc0, acc1 = lax.fori_loop(
                0, 12, accum_g,
                (jnp.zeros((16,), jnp.float32), jnp.zeros((16,), jnp.float32)))
            scalev = 1.0 / (3.0 * jnp.broadcast_to(cnt, (16,)))
            sessb[s, pl.ds(0, 16)] = acc0 * scalev
            sessb[s, pl.ds(16, 16)] = acc1 * scalev
            return 0
        lax.fori_loop(0, per_w, body, 0)
        pltpu.sync_copy(sessb, sout.at[pl.ds(b0, per_w)])

    return sess_k(h0, h1, h2, items.reshape(-1), inputs.reshape(-1),
                  alias_inputs.reshape(-1))


def _score_body(sess_ref, e0_ref, e1_ref, e2_ref, out_ref):
    sess = sess_ref[...]
    sn = jnp.sqrt(jnp.sum(sess * sess, axis=-1, keepdims=True))
    sel = _W_K * sess / jnp.maximum(sn, _EPS)
    emb = (e0_ref[...] + e1_ref[...] + e2_ref[...]) * (1.0 / 3.0)
    en = jnp.sqrt(jnp.sum(emb * emb, axis=-1, keepdims=True))
    embn = emb / jnp.maximum(en, _EPS)
    out_ref[...] = lax.dot_general(
        sel, embn, (((1,), (1,)), ((), ())),
        preferred_element_type=jnp.float32)


def _scores(sess_emb, h0, h1, h2, n):
    b, d = sess_emb.shape
    grid = _N_PAD // _BN
    return pl.pallas_call(
        _score_body,
        grid=(grid,),
        in_specs=[
            pl.BlockSpec((b, d), lambda i: (0, 0)),
            pl.BlockSpec((_BN, d), lambda i: (i, 0)),
            pl.BlockSpec((_BN, d), lambda i: (i, 0)),
            pl.BlockSpec((_BN, d), lambda i: (i, 0)),
        ],
        out_specs=pl.BlockSpec((b, _BN), lambda i: (0, i)),
        out_shape=jax.ShapeDtypeStruct((b, n), jnp.float32),
    )(sess_emb, h0, h1, h2)


def kernel(item_embedding, edge_val, items, inputs, alias_inputs, edge_row, edge_col):
    n = item_embedding.shape[0]
    h0 = jnp.pad(item_embedding, ((0, _N_PAD - n), (0, 0)))
    degrees = _degrees(edge_row, edge_val)
    h1 = _conv_layer(h0, degrees, edge_row, edge_col, edge_val)
    h2 = _conv_layer(h1, degrees, edge_row, edge_col, edge_val)
    sess_emb = _session_emb(h0, h1, h2, items, inputs, alias_inputs)
    return _scores(sess_emb, h0, h1, h2, n)
